# baseline (speedup 1.0000x reference)
import jax
import jax.numpy as jnp
from jax.experimental import pallas as pl
from jax.experimental.pallas import tpu as pltpu


_CONTRACT_SRC = (((0,), (0,)), ((), ()))
NCH = 2


def _mp_kernel(seq_hbm, adj_hbm, w_ref, ws_ref, b_ref, out_hbm):
    B, L, d = seq_hbm.shape
    CH = L // NCH

    def _body(seqf_ref, seqc_ref, adj_ref, out_ref):
        x = seqf_ref[0]
        a = (adj_ref[0] != 0).astype(jnp.float32)
        h = jnp.dot(x, w_ref[...], preferred_element_type=jnp.float32)
        agg = jax.lax.dot_general(
            a, h, _CONTRACT_SRC, preferred_element_type=jnp.float32
        )
        self_term = jnp.dot(
            seqc_ref[0], ws_ref[...], preferred_element_type=jnp.float32
        )
        out_ref[0] = agg + self_term + b_ref[...]

    pipeline = pltpu.emit_pipeline(
        _body,
        grid=(B, NCH),
        in_specs=[
            pl.BlockSpec((1, L, d), lambda g, c: (g, 0, 0)),
            pl.BlockSpec((1, CH, d), lambda g, c: (g, c, 0)),
            pl.BlockSpec((1, L, CH), lambda g, c: (g, 0, c)),
        ],
        out_specs=[
            pl.BlockSpec((1, CH, d), lambda g, c: (g, c, 0)),
        ],
    )
    pipeline(seq_hbm, seq_hbm, adj_hbm, out_hbm)


def kernel(seq, mask, adj_matrix, W, W_self, b):
    B, L, d = seq.shape
    del mask
    b2d = b.reshape(1, d)
    out = pl.pallas_call(
        _mp_kernel,
        in_specs=[
            pl.BlockSpec(memory_space=pl.ANY),
            pl.BlockSpec(memory_space=pl.ANY),
            pl.BlockSpec(memory_space=pltpu.VMEM),
            pl.BlockSpec(memory_space=pltpu.VMEM),
            pl.BlockSpec(memory_space=pltpu.VMEM),
        ],
        out_specs=pl.BlockSpec(memory_space=pl.ANY),
        out_shape=jax.ShapeDtypeStruct((B, L, d), jnp.float32),
    )(seq, adj_matrix, W, W_self, b2d)
    return out


# speedup vs baseline: 2.0081x; 2.0081x over previous
import jax
import jax.numpy as jnp
from jax.experimental import pallas as pl


GB = 2

_CONTRACT_SRC = (((0,), (0,)), ((), ()))


def _mp_kernel(seq_ref, adj_ref, w_ref, ws_ref, b_ref, out_ref):
    for g in range(GB):
        x = seq_ref[g]
        a = (adj_ref[g] != 0).astype(jnp.float32)
        h = jnp.dot(x, w_ref[...], preferred_element_type=jnp.float32)
        agg = jax.lax.dot_general(
            a, h, _CONTRACT_SRC, preferred_element_type=jnp.float32
        )
        self_term = jnp.dot(x, ws_ref[...], preferred_element_type=jnp.float32)
        out_ref[g] = agg + self_term + b_ref[...]


def kernel(seq, mask, adj_matrix, W, W_self, b):
    B, L, d = seq.shape
    del mask
    b2d = b.reshape(1, d)
    out = pl.pallas_call(
        _mp_kernel,
        grid=(B // GB,),
        in_specs=[
            pl.BlockSpec((GB, L, d), lambda i: (i, 0, 0)),
            pl.BlockSpec((GB, L, L), lambda i: (i, 0, 0)),
            pl.BlockSpec((d, d), lambda i: (0, 0)),
            pl.BlockSpec((d, d), lambda i: (0, 0)),
            pl.BlockSpec((1, d), lambda i: (0, 0)),
        ],
        out_specs=pl.BlockSpec((GB, L, d), lambda i: (i, 0, 0)),
        out_shape=jax.ShapeDtypeStruct((B, L, d), jnp.float32),
    )(seq, adj_matrix, W, W_self, b2d)
    return out
